# baseline (speedup 1.0000x reference)
import functools

import jax
import jax.numpy as jnp
from jax import lax
from jax.experimental import pallas as pl
from jax.experimental.pallas import tpu as pltpu
from jax.experimental.pallas import tpu_sc as plsc

_N = 10000
_E = 160000
_EPAD = 160768
_NW = 32
_K = 16


def _mm_body(x_ref, w_ref, b_ref, o_ref):
    o_ref[...] = (
        jnp.dot(x_ref[...], w_ref[...], preferred_element_type=jnp.float32,
                precision=lax.Precision.HIGHEST)
        + b_ref[...]
    )


@functools.lru_cache(maxsize=None)
def _mm_fn(m, kd, d, bm):
    return jax.jit(pl.pallas_call(
        _mm_body,
        grid=(m // bm,),
        in_specs=[
            pl.BlockSpec((bm, kd), lambda i: (i, 0)),
            pl.BlockSpec((kd, d), lambda i: (0, 0)),
            pl.BlockSpec((1, d), lambda i: (0, 0)),
        ],
        out_specs=pl.BlockSpec((bm, d), lambda i: (i, 0)),
        out_shape=jax.ShapeDtypeStruct((m, d), jnp.float32),
    ))


def _mm(x, w, b, bm):
    m, kd = x.shape
    d = w.shape[1]
    return _mm_fn(m, kd, d, bm)(x, w, b.reshape(1, d))


def _comb_body(*refs):
    w_ref = refs[0]
    o_ref = refs[-1]
    terms = refs[1:-1]
    acc = w_ref[0] * terms[0][...]
    for j in range(1, len(terms)):
        acc = acc + w_ref[j] * terms[j][...]
    o_ref[...] = acc


@functools.lru_cache(maxsize=None)
def _combine_fn(n, m, d, bm):
    return jax.jit(pl.pallas_call(
        _comb_body,
        grid=(m // bm,),
        in_specs=[pl.BlockSpec(memory_space=pltpu.SMEM)]
        + [pl.BlockSpec((bm, d), lambda i: (i, 0)) for _ in range(n)],
        out_specs=pl.BlockSpec((bm, d), lambda i: (i, 0)),
        out_shape=jax.ShapeDtypeStruct((m, d), jnp.float32),
    ))


def _combine(terms, w, bm=400):
    m, d = terms[0].shape
    return _combine_fn(len(terms), m, d, bm)(w, *terms)


def _bcast_lane(v, e):
    idx = jnp.full((16, 1), e, dtype=jnp.int32)
    dn = lax.GatherDimensionNumbers(
        offset_dims=(), collapsed_slice_dims=(0,), start_index_map=(0,))
    return lax.gather(v, idx, dn, (1,),
                      mode=lax.GatherScatterMode.PROMISE_IN_BOUNDS)


@functools.lru_cache(maxsize=None)
def _edge_kernel(d, r, np_):
    nr = _NW * np_
    npad = nr * r
    nch = d // 16
    mesh = plsc.VectorSubcoreMesh(core_axis_name="c", subcore_axis_name="s")

    def body(h_hbm, c_hbm, src_hbm, dst_hbm, bnd_hbm, out_hbm,
             slab, g2, cb, sidx, dvec, bvm, sem):
        cid = lax.axis_index("c")
        sid = lax.axis_index("s")
        wid = sid * 2 + cid
        pltpu.sync_copy(bnd_hbm, bvm)
        iota = lax.iota(jnp.int32, 16)
        zeros16 = jnp.zeros((16,), jnp.float32)

        def bnd_at(j):
            v = bvm[pl.ds((j // 16) * 16, 16)]
            return jnp.max(jnp.where(iota == j % 16, v, jnp.int32(-2147483647)))
        for p in range(np_):
            rid = p * _NW + wid
            base = rid * r

            def zbody(i, _):
                for u in range(8):
                    slab[pl.ds((i * 8 + u) * 16, 16)] = zeros16
                return 0

            lax.fori_loop(0, (r + 1) * d // 128, zbody, 0)

            e_lo = bnd_at(rid)
            e_hi = bnd_at(rid + 1)
            e_start = (e_lo // 8) * 8
            n_chunks = (e_hi - e_start + (_K - 1)) // _K

            def cbody(ci, _):
                e0 = e_start + ci * _K
                pltpu.sync_copy(src_hbm.at[pl.ds(e0, _K)], sidx)
                pltpu.sync_copy(dst_hbm.at[pl.ds(e0, _K)], dvec)
                pltpu.async_copy(h_hbm.at[sidx], g2, sem).wait()
                pltpu.sync_copy(c_hbm.at[pl.ds(e0 * d, _K * d)], cb)
                dv = dvec[...]
                local = dv - base
                valid = (local >= 0) & (local < r)
                lrow = jnp.where(valid, local, r)
                fbase = lrow * d
                for e in range(_K):
                    bvec = _bcast_lane(fbase, e)
                    for ch in range(nch):
                        g = g2[e, pl.ds(ch * 16, 16)]
                        cc = cb[pl.ds(e * d + ch * 16, 16)]
                        m = jnp.maximum(g + cc, 0.0)
                        plsc.addupdate_scatter(
                            slab, [bvec + (iota + ch * 16)], m)
                return 0

            lax.fori_loop(0, n_chunks, cbody, 0)
            pltpu.sync_copy(slab.at[pl.ds(0, r * d)],
                            out_hbm.at[pl.ds(base * d, r * d)])

    return jax.jit(pl.kernel(
        body,
        out_type=jax.ShapeDtypeStruct((npad * d,), jnp.float32),
        mesh=mesh,
        scratch_types=[
            pltpu.VMEM(((r + 1) * d,), jnp.float32),
            pltpu.VMEM((_K, d), jnp.float32),
            pltpu.VMEM((_K * d,), jnp.float32),
            pltpu.VMEM((_K,), jnp.int32),
            pltpu.VMEM((_K,), jnp.int32),
            pltpu.VMEM((nr + 8,), jnp.int32),
            pltpu.SemaphoreType.DMA,
        ],
        compiler_params=pltpu.CompilerParams(needs_layout_passes=False),
    )), npad


def kernel(x, edge_index, edge_attr,
           Wn1, bn1, We1, be1, Wn2, bn2, We2, be2, Wn3, bn3, We3, be3,
           Wn4, bn4, We4, be4, Wn5, bn5, We5, be5, Wn6, bn6, We6, be6,
           Wn7, bn7, We7, be7, Wn8, bn8, We8, be8, WnL, bnL, WeL, beL,
           w2, w3, w4, w5, w6, w7, w8, w9):
    src = edge_index[0]
    dst = edge_index[1]
    perm = jnp.argsort(dst)
    dst_sorted = dst[perm]
    src_s = jnp.concatenate([src[perm], jnp.zeros((32,), jnp.int32)])
    dst_s = jnp.concatenate(
        [dst_sorted, jnp.full((32,), 1 << 29, jnp.int32)])
    ea_s = jnp.zeros((_EPAD, 16), jnp.float32).at[:_E].set(edge_attr[perm])

    def bounds(r, np_):
        nr = _NW * np_
        b = jnp.searchsorted(
            dst_sorted, jnp.arange(nr + 1, dtype=jnp.int32) * r).astype(jnp.int32)
        return jnp.concatenate([b, jnp.full((7,), _E, jnp.int32)])

    b512 = bounds(160, 2)
    b256 = bounds(320, 1)
    ek512, npad512 = _edge_kernel(512, 160, 2)
    ek256, npad256 = _edge_kernel(256, 320, 1)

    def layer(xin, Wn, bn, We, be):
        d = Wn.shape[1]
        h = _mm(xin, Wn, bn, 400)
        c = _mm(ea_s, We, be, 1024)
        ek, b = (ek512, b512) if d == 512 else (ek256, b256)
        o = ek(h, c.reshape(-1), src_s, dst_s, b)
        return o.reshape(-1, d)[:_N]

    x1 = layer(x, Wn1, bn1, We1, be1)
    x2 = layer(x1, Wn2, bn2, We2, be2)
    x2w = _combine([x1, x2], w2)
    x3 = layer(x2w, Wn3, bn3, We3, be3)
    x3w = _combine([x1, x2w, x3], w3)
    x4 = layer(x3w, Wn4, bn4, We4, be4)
    x4w = _combine([x1, x2w, x3w, x4], w4)
    x5 = layer(x4w, Wn4, bn4, We4, be4)
    x5w = _combine([x1, x2w, x3w, x4w, x5], w5)
    x6 = layer(x5w, Wn5, bn5, We5, be5)
    x6w = _combine([x1, x2w, x3w, x4w, x5w, x6], w6)
    x7 = layer(x6w, Wn6, bn6, We6, be6)
    x7w = _combine([x1, x2w, x3w, x4w, x5w, x6w, x7], w7)
    x8 = layer(x7w, Wn7, bn7, We7, be7)
    x8w = _combine([x1, x2w, x3w, x4w, x5w, x6w, x7w, x8], w8)
    x9 = layer(x8w, Wn8, bn8, We8, be8)
    x9w = _combine([x1, x2w, x3w, x4w, x5w, x6w, x7w, x8w, x9], w9)
    return layer(x9w, WnL, bnL, WeL, beL)


# speedup vs baseline: 1.9822x; 1.9822x over previous
import functools

import jax
import jax.numpy as jnp
from jax import lax
from jax.experimental import pallas as pl
from jax.experimental.pallas import tpu as pltpu
from jax.experimental.pallas import tpu_sc as plsc

_N = 10000
_E = 160000
_EPAD = 160768
_NW = 32
_K = 32
_PAD = 192


def _mm_body(x_ref, w_ref, b_ref, o_ref):
    o_ref[...] = (
        jnp.dot(x_ref[...], w_ref[...], preferred_element_type=jnp.float32,
                precision=lax.Precision.HIGHEST)
        + b_ref[...]
    )


@functools.lru_cache(maxsize=None)
def _mm_fn(m, kd, d, bm):
    return jax.jit(pl.pallas_call(
        _mm_body,
        grid=(m // bm,),
        in_specs=[
            pl.BlockSpec((bm, kd), lambda i: (i, 0)),
            pl.BlockSpec((kd, d), lambda i: (0, 0)),
            pl.BlockSpec((1, d), lambda i: (0, 0)),
        ],
        out_specs=pl.BlockSpec((bm, d), lambda i: (i, 0)),
        out_shape=jax.ShapeDtypeStruct((m, d), jnp.float32),
    ))


def _mm(x, w, b, bm):
    m, kd = x.shape
    d = w.shape[1]
    return _mm_fn(m, kd, d, bm)(x, w, b.reshape(1, d))


def _comb_body(*refs):
    w_ref = refs[0]
    o_ref = refs[-1]
    terms = refs[1:-1]
    acc = w_ref[0] * terms[0][...]
    for j in range(1, len(terms)):
        acc = acc + w_ref[j] * terms[j][...]
    o_ref[...] = acc


@functools.lru_cache(maxsize=None)
def _combine_fn(n, m, d, bm):
    return jax.jit(pl.pallas_call(
        _comb_body,
        grid=(m // bm,),
        in_specs=[pl.BlockSpec(memory_space=pltpu.SMEM)]
        + [pl.BlockSpec((bm, d), lambda i: (i, 0)) for _ in range(n)],
        out_specs=pl.BlockSpec((bm, d), lambda i: (i, 0)),
        out_shape=jax.ShapeDtypeStruct((m, d), jnp.float32),
    ))


def _combine(terms, w, bm=400):
    m, d = terms[0].shape
    return _combine_fn(len(terms), m, d, bm)(w, *terms)


def _bcast_lane(v, e):
    idx = jnp.full((16, 1), e, dtype=jnp.int32)
    dn = lax.GatherDimensionNumbers(
        offset_dims=(), collapsed_slice_dims=(0,), start_index_map=(0,))
    return lax.gather(v, idx, dn, (1,),
                      mode=lax.GatherScatterMode.PROMISE_IN_BOUNDS)


@functools.lru_cache(maxsize=None)
def _edge_kernel(d, r, np_):
    nr = _NW * np_
    npad = nr * r
    nchd = d // 16
    mesh = plsc.VectorSubcoreMesh(core_axis_name="c", subcore_axis_name="s")

    def body(h_hbm, c_hbm, src_hbm, dst_hbm, bnd_hbm, out_hbm,
             slab, g0, g1, cb0, cb1, si0, si1, dv0, dv1, bvm,
             sg0, sg1, sc0, sc1, ssi0, ssi1, sdi0, sdi1):
        cid = lax.axis_index("c")
        sid = lax.axis_index("s")
        wid = sid * 2 + cid
        pltpu.sync_copy(bnd_hbm, bvm)
        iota = lax.iota(jnp.int32, 16)
        zeros16 = jnp.zeros((16,), jnp.float32)

        def bnd_at(j):
            v = bvm[pl.ds((j // 16) * 16, 16)]
            return jnp.max(jnp.where(iota == j % 16, v, jnp.int32(-2147483647)))

        def start_idx(e0, si, dvv, ssi, sdi):
            pltpu.async_copy(src_hbm.at[pl.ds(e0, _K)], si, ssi)
            pltpu.async_copy(dst_hbm.at[pl.ds(e0, _K)], dvv, sdi)

        def wait_idx(si, dvv, ssi, sdi):
            pltpu.make_async_copy(src_hbm.at[pl.ds(0, _K)], si, ssi).wait()
            pltpu.make_async_copy(dst_hbm.at[pl.ds(0, _K)], dvv, sdi).wait()

        def start_gc(e0, si, g, cb, sg, sc):
            pltpu.async_copy(h_hbm.at[si], g, sg)
            pltpu.async_copy(c_hbm.at[pl.ds(e0 * d, _K * d)], cb, sc)

        def wait_gc(si, g, cb, sg, sc):
            pltpu.make_async_copy(h_hbm.at[si], g, sg).wait()
            pltpu.make_async_copy(c_hbm.at[pl.ds(0, _K * d)], cb, sc).wait()

        for p in range(np_):
            rid = p * _NW + wid
            base = rid * r

            def zbody(i, _):
                for u in range(8):
                    slab[pl.ds((i * 8 + u) * 16, 16)] = zeros16
                return 0

            lax.fori_loop(0, (r + 1) * d // 128, zbody, 0)

            e_lo = bnd_at(rid)
            e_hi = bnd_at(rid + 1)
            e_start = (e_lo // 8) * 8
            n_chunks = (e_hi - e_start + (_K - 1)) // _K

            def compute(g, cb, dva, dvb):
                for half, dvh in enumerate((dva, dvb)):
                    local = dvh - base
                    valid = (local >= 0) & (local < r)
                    lrow = jnp.where(valid, local, r)
                    fbase = lrow * d
                    bvecs = [_bcast_lane(fbase, e) for e in range(16)]

                    def chbody(ch, _):
                        off = ch * 16
                        for e in range(16):
                            ea = half * 16 + e
                            gv = g[ea, pl.ds(off, 16)]
                            cc = cb[pl.ds(ea * d + off, 16)]
                            m = jnp.maximum(gv + cc, 0.0)
                            plsc.addupdate_scatter(
                                slab, [bvecs[e] + (off + iota)], m)
                        return 0

                    lax.fori_loop(0, nchd, chbody, 0, unroll=4)

            pltpu.sync_copy(src_hbm.at[pl.ds(e_start, _K)], si0)
            pltpu.sync_copy(dst_hbm.at[pl.ds(e_start, _K)], dv0)
            start_gc(e_start, si0, g0, cb0, sg0, sc0)
            start_idx(e_start + _K, si1, dv1, ssi1, sdi1)

            def pair(j, _):
                e0 = e_start + (2 * j) * _K
                wait_gc(si0, g0, cb0, sg0, sc0)
                wait_idx(si1, dv1, ssi1, sdi1)
                start_gc(e0 + _K, si1, g1, cb1, sg1, sc1)
                dva = dv0[pl.ds(0, 16)]
                dvb = dv0[pl.ds(16, 16)]
                start_idx(e0 + 2 * _K, si0, dv0, ssi0, sdi0)
                compute(g0, cb0, dva, dvb)
                wait_gc(si1, g1, cb1, sg1, sc1)
                wait_idx(si0, dv0, ssi0, sdi0)
                start_gc(e0 + 2 * _K, si0, g0, cb0, sg0, sc0)
                dva = dv1[pl.ds(0, 16)]
                dvb = dv1[pl.ds(16, 16)]
                start_idx(e0 + 3 * _K, si1, dv1, ssi1, sdi1)
                compute(g1, cb1, dva, dvb)
                return 0

            lax.fori_loop(0, (n_chunks + 1) // 2, pair, 0)
            wait_gc(si0, g0, cb0, sg0, sc0)
            wait_idx(si1, dv1, ssi1, sdi1)
            pltpu.sync_copy(slab.at[pl.ds(0, r * d)],
                            out_hbm.at[pl.ds(base * d, r * d)])

    return jax.jit(pl.kernel(
        body,
        out_type=jax.ShapeDtypeStruct((npad * d,), jnp.float32),
        mesh=mesh,
        scratch_types=[
            pltpu.VMEM(((r + 1) * d,), jnp.float32),
            pltpu.VMEM((_K, d), jnp.float32),
            pltpu.VMEM((_K, d), jnp.float32),
            pltpu.VMEM((_K * d,), jnp.float32),
            pltpu.VMEM((_K * d,), jnp.float32),
            pltpu.VMEM((_K,), jnp.int32),
            pltpu.VMEM((_K,), jnp.int32),
            pltpu.VMEM((_K,), jnp.int32),
            pltpu.VMEM((_K,), jnp.int32),
            pltpu.VMEM((nr + 8,), jnp.int32),
            pltpu.SemaphoreType.DMA, pltpu.SemaphoreType.DMA,
            pltpu.SemaphoreType.DMA, pltpu.SemaphoreType.DMA,
            pltpu.SemaphoreType.DMA, pltpu.SemaphoreType.DMA,
            pltpu.SemaphoreType.DMA, pltpu.SemaphoreType.DMA,
        ],
        compiler_params=pltpu.CompilerParams(needs_layout_passes=False),
    )), npad


def kernel(x, edge_index, edge_attr,
           Wn1, bn1, We1, be1, Wn2, bn2, We2, be2, Wn3, bn3, We3, be3,
           Wn4, bn4, We4, be4, Wn5, bn5, We5, be5, Wn6, bn6, We6, be6,
           Wn7, bn7, We7, be7, Wn8, bn8, We8, be8, WnL, bnL, WeL, beL,
           w2, w3, w4, w5, w6, w7, w8, w9):
    src = edge_index[0]
    dst = edge_index[1]
    perm = jnp.argsort(dst)
    dst_sorted = dst[perm]
    src_s = jnp.concatenate([src[perm], jnp.zeros((_PAD,), jnp.int32)])
    dst_s = jnp.concatenate(
        [dst_sorted, jnp.full((_PAD,), 1 << 29, jnp.int32)])
    ea_s = jnp.zeros((_EPAD, 16), jnp.float32).at[:_E].set(edge_attr[perm])

    def bounds(r, np_):
        nr = _NW * np_
        b = jnp.searchsorted(
            dst_sorted, jnp.arange(nr + 1, dtype=jnp.int32) * r).astype(jnp.int32)
        return jnp.concatenate([b, jnp.full((7,), _E, jnp.int32)])

    b512 = bounds(120, 3)
    b256 = bounds(320, 1)
    ek512, npad512 = _edge_kernel(512, 120, 3)
    ek256, npad256 = _edge_kernel(256, 320, 1)

    def layer(xin, Wn, bn, We, be):
        d = Wn.shape[1]
        h = _mm(xin, Wn, bn, 400)
        c = _mm(ea_s, We, be, 1024)
        ek, b = (ek512, b512) if d == 512 else (ek256, b256)
        o = ek(h, c.reshape(-1), src_s, dst_s, b)
        return o.reshape(-1, d)[:_N]

    x1 = layer(x, Wn1, bn1, We1, be1)
    x2 = layer(x1, Wn2, bn2, We2, be2)
    x2w = _combine([x1, x2], w2)
    x3 = layer(x2w, Wn3, bn3, We3, be3)
    x3w = _combine([x1, x2w, x3], w3)
    x4 = layer(x3w, Wn4, bn4, We4, be4)
    x4w = _combine([x1, x2w, x3w, x4], w4)
    x5 = layer(x4w, Wn4, bn4, We4, be4)
    x5w = _combine([x1, x2w, x3w, x4w, x5], w5)
    x6 = layer(x5w, Wn5, bn5, We5, be5)
    x6w = _combine([x1, x2w, x3w, x4w, x5w, x6], w6)
    x7 = layer(x6w, Wn6, bn6, We6, be6)
    x7w = _combine([x1, x2w, x3w, x4w, x5w, x6w, x7], w7)
    x8 = layer(x7w, Wn7, bn7, We7, be7)
    x8w = _combine([x1, x2w, x3w, x4w, x5w, x6w, x7w, x8], w8)
    x9 = layer(x8w, Wn8, bn8, We8, be8)
    x9w = _combine([x1, x2w, x3w, x4w, x5w, x6w, x7w, x8w, x9], w9)
    return layer(x9w, WnL, bnL, WeL, beL)


# speedup vs baseline: 1.9974x; 1.0077x over previous
import functools

import jax
import jax.numpy as jnp
from jax import lax
from jax.experimental import pallas as pl
from jax.experimental.pallas import tpu as pltpu
from jax.experimental.pallas import tpu_sc as plsc

_N = 10000
_E = 160000
_EPAD = 160768
_NW = 32
_K = 32
_PAD = 192


def _mm_body(x_ref, w_ref, b_ref, o_ref):
    o_ref[...] = (
        jnp.dot(x_ref[...], w_ref[...], preferred_element_type=jnp.float32)
        + b_ref[...]
    )


@functools.lru_cache(maxsize=None)
def _mm_fn(m, kd, d, bm):
    return jax.jit(pl.pallas_call(
        _mm_body,
        grid=(m // bm,),
        in_specs=[
            pl.BlockSpec((bm, kd), lambda i: (i, 0)),
            pl.BlockSpec((kd, d), lambda i: (0, 0)),
            pl.BlockSpec((1, d), lambda i: (0, 0)),
        ],
        out_specs=pl.BlockSpec((bm, d), lambda i: (i, 0)),
        out_shape=jax.ShapeDtypeStruct((m, d), jnp.float32),
    ))


def _mm(x, w, b, bm):
    m, kd = x.shape
    d = w.shape[1]
    return _mm_fn(m, kd, d, bm)(x, w, b.reshape(1, d))


def _comb_body(*refs):
    w_ref = refs[0]
    o_ref = refs[-1]
    terms = refs[1:-1]
    acc = w_ref[0] * terms[0][...]
    for j in range(1, len(terms)):
        acc = acc + w_ref[j] * terms[j][...]
    o_ref[...] = acc


@functools.lru_cache(maxsize=None)
def _combine_fn(n, m, d, bm):
    return jax.jit(pl.pallas_call(
        _comb_body,
        grid=(m // bm,),
        in_specs=[pl.BlockSpec(memory_space=pltpu.SMEM)]
        + [pl.BlockSpec((bm, d), lambda i: (i, 0)) for _ in range(n)],
        out_specs=pl.BlockSpec((bm, d), lambda i: (i, 0)),
        out_shape=jax.ShapeDtypeStruct((m, d), jnp.float32),
    ))


def _combine(terms, w, bm=400):
    m, d = terms[0].shape
    return _combine_fn(len(terms), m, d, bm)(w, *terms)


def _bcast_lane(v, e):
    idx = jnp.full((16, 1), e, dtype=jnp.int32)
    dn = lax.GatherDimensionNumbers(
        offset_dims=(), collapsed_slice_dims=(0,), start_index_map=(0,))
    return lax.gather(v, idx, dn, (1,),
                      mode=lax.GatherScatterMode.PROMISE_IN_BOUNDS)


@functools.lru_cache(maxsize=None)
def _edge_kernel(d, r, np_):
    nr = _NW * np_
    npad = nr * r
    nchd = d // 16
    mesh = plsc.VectorSubcoreMesh(core_axis_name="c", subcore_axis_name="s")

    def body(h_hbm, c_hbm, src_hbm, dst_hbm, bnd_hbm, out_hbm,
             slab, g0, g1, cb0, cb1, si0, si1, dv0, dv1, bvm,
             sg0, sg1, sc0, sc1, ssi0, ssi1, sdi0, sdi1):
        cid = lax.axis_index("c")
        sid = lax.axis_index("s")
        wid = sid * 2 + cid
        pltpu.sync_copy(bnd_hbm, bvm)
        iota = lax.iota(jnp.int32, 16)
        zeros16 = jnp.zeros((16,), jnp.float32)

        def bnd_at(j):
            v = bvm[pl.ds((j // 16) * 16, 16)]
            return jnp.max(jnp.where(iota == j % 16, v, jnp.int32(-2147483647)))

        def start_idx(e0, si, dvv, ssi, sdi):
            pltpu.async_copy(src_hbm.at[pl.ds(e0, _K)], si, ssi)
            pltpu.async_copy(dst_hbm.at[pl.ds(e0, _K)], dvv, sdi)

        def wait_idx(si, dvv, ssi, sdi):
            pltpu.make_async_copy(src_hbm.at[pl.ds(0, _K)], si, ssi).wait()
            pltpu.make_async_copy(dst_hbm.at[pl.ds(0, _K)], dvv, sdi).wait()

        def start_gc(e0, si, g, cb, sg, sc):
            pltpu.async_copy(h_hbm.at[si], g, sg)
            pltpu.async_copy(c_hbm.at[pl.ds(e0 * d, _K * d)], cb, sc)

        def wait_gc(si, g, cb, sg, sc):
            pltpu.make_async_copy(h_hbm.at[si], g, sg).wait()
            pltpu.make_async_copy(c_hbm.at[pl.ds(0, _K * d)], cb, sc).wait()

        for p in range(np_):
            rid = p * _NW + wid
            base = rid * r

            def zbody(i, _):
                for u in range(8):
                    slab[pl.ds((i * 8 + u) * 16, 16)] = zeros16
                return 0

            lax.fori_loop(0, (r + 1) * d // 128, zbody, 0)

            e_lo = bnd_at(rid)
            e_hi = bnd_at(rid + 1)
            e_start = (e_lo // 8) * 8
            n_chunks = (e_hi - e_start + (_K - 1)) // _K

            def compute(g, cb, dva, dvb):
                for half, dvh in enumerate((dva, dvb)):
                    local = dvh - base
                    valid = (local >= 0) & (local < r)
                    lrow = jnp.where(valid, local, r)
                    fbase = lrow * d
                    bvecs = [_bcast_lane(fbase, e) for e in range(16)]

                    def chbody(ch, _):
                        off = ch * 16
                        for e in range(16):
                            ea = half * 16 + e
                            gv = g[ea, pl.ds(off, 16)]
                            cc = cb[pl.ds(ea * d + off, 16)]
                            m = jnp.maximum(gv + cc, 0.0)
                            plsc.addupdate_scatter(
                                slab, [bvecs[e] + (off + iota)], m)
                        return 0

                    lax.fori_loop(0, nchd, chbody, 0, unroll=4)

            pltpu.sync_copy(src_hbm.at[pl.ds(e_start, _K)], si0)
            pltpu.sync_copy(dst_hbm.at[pl.ds(e_start, _K)], dv0)
            start_gc(e_start, si0, g0, cb0, sg0, sc0)
            start_idx(e_start + _K, si1, dv1, ssi1, sdi1)

            def pair(j, _):
                e0 = e_start + (2 * j) * _K
                wait_gc(si0, g0, cb0, sg0, sc0)
                wait_idx(si1, dv1, ssi1, sdi1)
                start_gc(e0 + _K, si1, g1, cb1, sg1, sc1)
                dva = dv0[pl.ds(0, 16)]
                dvb = dv0[pl.ds(16, 16)]
                start_idx(e0 + 2 * _K, si0, dv0, ssi0, sdi0)
                compute(g0, cb0, dva, dvb)
                wait_gc(si1, g1, cb1, sg1, sc1)
                wait_idx(si0, dv0, ssi0, sdi0)
                start_gc(e0 + 2 * _K, si0, g0, cb0, sg0, sc0)
                dva = dv1[pl.ds(0, 16)]
                dvb = dv1[pl.ds(16, 16)]
                start_idx(e0 + 3 * _K, si1, dv1, ssi1, sdi1)
                compute(g1, cb1, dva, dvb)
                return 0

            lax.fori_loop(0, (n_chunks + 1) // 2, pair, 0)
            wait_gc(si0, g0, cb0, sg0, sc0)
            wait_idx(si1, dv1, ssi1, sdi1)
            pltpu.sync_copy(slab.at[pl.ds(0, r * d)],
                            out_hbm.at[pl.ds(base * d, r * d)])

    return jax.jit(pl.kernel(
        body,
        out_type=jax.ShapeDtypeStruct((npad * d,), jnp.float32),
        mesh=mesh,
        scratch_types=[
            pltpu.VMEM(((r + 1) * d,), jnp.float32),
            pltpu.VMEM((_K, d), jnp.float32),
            pltpu.VMEM((_K, d), jnp.float32),
            pltpu.VMEM((_K * d,), jnp.float32),
            pltpu.VMEM((_K * d,), jnp.float32),
            pltpu.VMEM((_K,), jnp.int32),
            pltpu.VMEM((_K,), jnp.int32),
            pltpu.VMEM((_K,), jnp.int32),
            pltpu.VMEM((_K,), jnp.int32),
            pltpu.VMEM((nr + 8,), jnp.int32),
            pltpu.SemaphoreType.DMA, pltpu.SemaphoreType.DMA,
            pltpu.SemaphoreType.DMA, pltpu.SemaphoreType.DMA,
            pltpu.SemaphoreType.DMA, pltpu.SemaphoreType.DMA,
            pltpu.SemaphoreType.DMA, pltpu.SemaphoreType.DMA,
        ],
        compiler_params=pltpu.CompilerParams(needs_layout_passes=False),
    )), npad


def kernel(x, edge_index, edge_attr,
           Wn1, bn1, We1, be1, Wn2, bn2, We2, be2, Wn3, bn3, We3, be3,
           Wn4, bn4, We4, be4, Wn5, bn5, We5, be5, Wn6, bn6, We6, be6,
           Wn7, bn7, We7, be7, Wn8, bn8, We8, be8, WnL, bnL, WeL, beL,
           w2, w3, w4, w5, w6, w7, w8, w9):
    src = edge_index[0]
    dst = edge_index[1]
    perm = jnp.argsort(dst)
    dst_sorted = dst[perm]
    src_s = jnp.concatenate([src[perm], jnp.zeros((_PAD,), jnp.int32)])
    dst_s = jnp.concatenate(
        [dst_sorted, jnp.full((_PAD,), 1 << 29, jnp.int32)])
    ea_s = jnp.zeros((_EPAD, 16), jnp.float32).at[:_E].set(edge_attr[perm])

    def bounds(r, np_):
        nr = _NW * np_
        b = jnp.searchsorted(
            dst_sorted, jnp.arange(nr + 1, dtype=jnp.int32) * r).astype(jnp.int32)
        return jnp.concatenate([b, jnp.full((7,), _E, jnp.int32)])

    b512 = bounds(120, 3)
    b256 = bounds(320, 1)
    ek512, npad512 = _edge_kernel(512, 120, 3)
    ek256, npad256 = _edge_kernel(256, 320, 1)

    def layer(xin, Wn, bn, We, be):
        d = Wn.shape[1]
        h = _mm(xin, Wn, bn, 400)
        c = _mm(ea_s, We, be, 1024)
        ek, b = (ek512, b512) if d == 512 else (ek256, b256)
        o = ek(h, c.reshape(-1), src_s, dst_s, b)
        return o.reshape(-1, d)[:_N]

    x1 = layer(x, Wn1, bn1, We1, be1)
    x2 = layer(x1, Wn2, bn2, We2, be2)
    x2w = _combine([x1, x2], w2)
    x3 = layer(x2w, Wn3, bn3, We3, be3)
    x3w = _combine([x1, x2w, x3], w3)
    x4 = layer(x3w, Wn4, bn4, We4, be4)
    x4w = _combine([x1, x2w, x3w, x4], w4)
    x5 = layer(x4w, Wn4, bn4, We4, be4)
    x5w = _combine([x1, x2w, x3w, x4w, x5], w5)
    x6 = layer(x5w, Wn5, bn5, We5, be5)
    x6w = _combine([x1, x2w, x3w, x4w, x5w, x6], w6)
    x7 = layer(x6w, Wn6, bn6, We6, be6)
    x7w = _combine([x1, x2w, x3w, x4w, x5w, x6w, x7], w7)
    x8 = layer(x7w, Wn7, bn7, We7, be7)
    x8w = _combine([x1, x2w, x3w, x4w, x5w, x6w, x7w, x8], w8)
    x9 = layer(x8w, Wn8, bn8, We8, be8)
    x9w = _combine([x1, x2w, x3w, x4w, x5w, x6w, x7w, x8w, x9], w9)
    return layer(x9w, WnL, bnL, WeL, beL)
